# baseline (speedup 1.0000x reference)
import functools
import math

import jax
import jax.numpy as jnp
from jax import lax
from jax.experimental import pallas as pl
from jax.experimental.pallas import tpu as pltpu
from jax.experimental.pallas import tpu_sc as plsc

V = 32000
N = 4096
_SMOOTH = 0.1 / (V - 2)
_CONF = 0.9
_C_ENT = (V - 2) * _SMOOTH * math.log(_SMOOTH) + _CONF * math.log(_CONF)

_CSC = 3200
_VTC = V - _CSC
_BC = 1152

_NC, _NS = 2, 16
_NW = _NC * _NS
_SPW = (N // 8) // _NW
_L = 16
_CB = _CSC // _L


def _loss_body(x_ref, t_ref, out_ref):
    j = pl.program_id(0)
    x = x_ref[...]
    t = t_ref[...]
    validf = jnp.where(t != 0, 1.0, 0.0)
    col = jax.lax.broadcasted_iota(jnp.int32, (N, _BC), 1)
    w = jnp.where(col == t - j * _BC, _CONF, _SMOOTH)
    rowterm = jnp.sum(x * w, axis=1, keepdims=True)
    partial = -jnp.sum(validf * rowterm, keepdims=True)

    @pl.when(j == 0)
    def _init():
        out_ref[...] = (_C_ENT * jnp.sum(validf, keepdims=True)
                        + _SMOOTH * jnp.sum(validf * x[:, 0:1], keepdims=True))

    out_ref[...] += partial


def _sc_body(x_hbm, t_hbm, out_hbm, t_v, buf_v, acc_v, sem0, sem1):
    wid = lax.axis_index("s") * _NC + lax.axis_index("c")
    row0 = wid * (_SPW * 8)
    pltpu.sync_copy(t_hbm.at[pl.ds(row0, _SPW * 8)], t_v)
    accA = jnp.zeros((_L,), jnp.float32)
    accB = jnp.zeros((_L,), jnp.float32)
    lane = lax.iota(jnp.int32, _L)
    lane0 = lane == 0
    sems = (sem0, sem1)
    cps = [None, None]
    cps[0] = pltpu.async_copy(
        x_hbm.at[pl.ds(row0, 8), pl.ds(_VTC, _CSC)], buf_v.at[0], sems[0])
    for k in range(_SPW):
        b = k % 2
        cps[b].wait()
        if k + 1 < _SPW:
            nb = (k + 1) % 2
            cps[nb] = pltpu.async_copy(
                x_hbm.at[pl.ds(row0 + (k + 1) * 8, 8), pl.ds(_VTC, _CSC)],
                buf_v.at[nb], sems[nb])
        for r in range(8):
            rl = k * 8 + r
            t16 = plsc.load_gather(t_v, [jnp.full((_L,), rl, jnp.int32)])
            w16 = jnp.where(t16 == 0, 0.0, 1.0)

            def chunk_body(c, racc, _b=b, _r=r):
                base = c * (8 * _L)
                for u in range(8):
                    racc = racc + buf_v[_b, _r, pl.ds(base + u * _L, _L)]
                return racc

            racc = lax.fori_loop(0, _CB // 8, chunk_body,
                                 jnp.zeros((_L,), jnp.float32))
            accA = accA + racc * w16
            toff = t16 - _VTC
            idxc = jnp.minimum(jnp.maximum(toff, 0), _CSC - 1)
            g16 = plsc.load_gather(
                buf_v, [jnp.full((_L,), b, jnp.int32),
                        jnp.full((_L,), r, jnp.int32), idxc])
            hit = jnp.logical_and(toff == idxc, lane0)
            accB = accB + jnp.where(hit, g16, 0.0) * w16
    acc_v[...] = _SMOOTH * accA + (_CONF - _SMOOTH) * accB
    pltpu.sync_copy(acc_v, out_hbm.at[wid])


_sc_kernel = functools.partial(
    pl.kernel,
    out_type=jax.ShapeDtypeStruct((_NW, _L), jnp.float32),
    mesh=plsc.VectorSubcoreMesh(
        core_axis_name="c", subcore_axis_name="s",
        num_cores=_NC, num_subcores=_NS),
    compiler_params=pltpu.CompilerParams(
        use_tc_tiling_on_sc=True, needs_layout_passes=False),
    scratch_types=[
        pltpu.VMEM((_SPW * 8,), jnp.int32),
        pltpu.VMEM((2, 8, _CSC), jnp.float32),
        pltpu.VMEM((_L,), jnp.float32),
        pltpu.SemaphoreType.DMA,
        pltpu.SemaphoreType.DMA,
    ],
)(_sc_body)


def kernel(output, target):
    t32 = target.astype(jnp.int32)
    t2 = t32.reshape(N, 1)
    sweep = pl.pallas_call(
        _loss_body,
        grid=(_VTC // _BC,),
        in_specs=[
            pl.BlockSpec((N, _BC), lambda j: (0, j)),
            pl.BlockSpec((N, 1), lambda j: (0, 0)),
        ],
        out_specs=pl.BlockSpec((1, 1), lambda j: (0, 0)),
        out_shape=jax.ShapeDtypeStruct((1, 1), jnp.float32),
    )(output, t2)
    partials = _sc_kernel(output, t32)
    return sweep[0, 0] - jnp.sum(partials)


# speedup vs baseline: 1.1234x; 1.1234x over previous
import math

import jax
import jax.numpy as jnp
from jax.experimental import pallas as pl

V = 32000
N = 4096
_SMOOTH = 0.1 / (V - 2)
_CONF = 0.9
_C_ENT = (V - 2) * _SMOOTH * math.log(_SMOOTH) + _CONF * math.log(_CONF)

_BC = 1280


def _loss_body(x_ref, t_ref, out_ref):
    j = pl.program_id(0)
    x = x_ref[...]
    t = t_ref[...]
    validf = jnp.where(t != 0, 1.0, 0.0)
    col = jax.lax.broadcasted_iota(jnp.int32, (N, _BC), 1)
    w = jnp.where(col == t - j * _BC, _CONF, _SMOOTH)
    rowterm = jnp.sum(x * w, axis=1, keepdims=True)
    partial = -jnp.sum(validf * rowterm, keepdims=True)

    @pl.when(j == 0)
    def _init():
        out_ref[...] = (_C_ENT * jnp.sum(validf, keepdims=True)
                        + _SMOOTH * jnp.sum(validf * x[:, 0:1], keepdims=True))

    out_ref[...] += partial


def kernel(output, target):
    t2 = target.astype(jnp.int32).reshape(N, 1)
    res = pl.pallas_call(
        _loss_body,
        grid=(V // _BC,),
        in_specs=[
            pl.BlockSpec((N, _BC), lambda j: (0, j)),
            pl.BlockSpec((N, 1), lambda j: (0, 0)),
        ],
        out_specs=pl.BlockSpec((1, 1), lambda j: (0, 0)),
        out_shape=jax.ShapeDtypeStruct((1, 1), jnp.float32),
    )(output, t2)
    return res[0, 0]
